# baseline (speedup 1.0000x reference)
import functools

import jax
import jax.numpy as jnp
from jax import lax
from jax.experimental import pallas as pl
from jax.experimental.pallas import tpu as pltpu
from jax.experimental.pallas import tpu_sc as plsc

_LANES = 16
_IGNORE_INDEX = -100


@functools.lru_cache(maxsize=None)
def _make_nll_kernel(B: int, C: int):
    num_subcores = 16
    b_per_w = B // num_subcores
    chunks = b_per_w // _LANES
    mesh = plsc.VectorSubcoreMesh(
        core_axis_name="c", subcore_axis_name="s", num_cores=1
    )

    @functools.partial(
        pl.kernel,
        out_type=jax.ShapeDtypeStruct((_LANES,), jnp.float32),
        mesh=mesh,
        compiler_params=pltpu.CompilerParams(needs_layout_passes=False),
        scratch_types=[
            pltpu.VMEM((b_per_w,), jnp.int32),
            pltpu.VMEM((b_per_w,), jnp.int32),
            pltpu.VMEM((b_per_w, 8, 128), jnp.float32),
            pltpu.VMEM((2 * _LANES,), jnp.float32),
            pltpu.VMEM_SHARED((num_subcores * 2 * _LANES,), jnp.float32),
            pltpu.VMEM((num_subcores * 2 * _LANES,), jnp.float32),
            pltpu.VMEM((_LANES,), jnp.float32),
            pltpu.VMEM((_LANES,), jnp.float32),
            pltpu.SemaphoreType.DMA,
        ],
    )
    def nll_kernel(predsT_hbm, tgt_hbm, out_hbm,
                   tgt_v, idx_v, slab_v, part_v, shared, all_v, res_v,
                   bfly_v, sem):
        sid = lax.axis_index("s")
        base = sid * b_per_w

        pltpu.sync_copy(tgt_hbm.at[pl.ds(base, b_per_w)], tgt_v)

        lane = lax.iota(jnp.int32, _LANES)
        for j in range(chunks):
            t = tgt_v[pl.ds(j * _LANES, _LANES)]
            safe = jnp.minimum(jnp.maximum(t, 0), C - 1)
            idx_v[pl.ds(j * _LANES, _LANES)] = safe >> 3

        c0 = pl.multiple_of((base >> 7) << 7, 128)
        view3 = predsT_hbm.reshape(C // 8, 8, B)
        copies = [
            pltpu.async_copy(
                view3.at[idx_v.at[pl.ds(j * _LANES, _LANES)], :,
                         pl.ds(c0, 128)],
                slab_v.at[pl.ds(j * _LANES, _LANES)],
                sem,
            )
            for j in range(chunks)
        ]

        acc = jnp.zeros((_LANES,), jnp.float32)
        cnt = jnp.zeros((_LANES,), jnp.float32)
        for j in range(chunks):
            copies[j].wait()
            t = tgt_v[pl.ds(j * _LANES, _LANES)]
            valid = t != _IGNORE_INDEX
            safe = jnp.minimum(jnp.maximum(t, 0), C - 1)
            sample = j * _LANES + lane
            colw = (base + sample) & 127
            v = plsc.load_gather(slab_v, [sample, safe & 7, colw])
            acc = acc + jnp.where(valid, -v, 0.0)
            cnt = cnt + jnp.where(valid, 1.0, 0.0)

        part_v[pl.ds(0, _LANES)] = acc
        part_v[pl.ds(_LANES, _LANES)] = cnt
        pltpu.sync_copy(part_v, shared.at[pl.ds(sid * 2 * _LANES, 2 * _LANES)])
        plsc.subcore_barrier()

        pltpu.sync_copy(shared, all_v)
        tot = jnp.zeros((_LANES,), jnp.float32)
        num = jnp.zeros((_LANES,), jnp.float32)
        for w in range(num_subcores):
            tot = tot + all_v[pl.ds(w * 2 * _LANES, _LANES)]
            num = num + all_v[pl.ds(w * 2 * _LANES + _LANES, _LANES)]

        def lane_sum(vec):
            for shift in (8, 4, 2, 1):
                bfly_v[...] = vec
                vec = vec + plsc.load_gather(bfly_v, [lane ^ shift])
            return vec

        s = lane_sum(tot)
        n = lane_sum(num)
        res_v[...] = s / jnp.maximum(n, 1.0)

        @pl.when(sid == 0)
        def _():
            pltpu.sync_copy(res_v, out_hbm)

    return nll_kernel


def kernel(predictions, targets):
    B, C = predictions.shape
    tgt = targets.astype(jnp.int32)
    out = _make_nll_kernel(B, C)(predictions.T, tgt)
    return out[0]


# speedup vs baseline: 1.0042x; 1.0042x over previous
import functools

import jax
import jax.numpy as jnp
from jax import lax
from jax.experimental import pallas as pl
from jax.experimental.pallas import tpu as pltpu
from jax.experimental.pallas import tpu_sc as plsc

_LANES = 16
_IGNORE_INDEX = -100


@functools.lru_cache(maxsize=None)
def _make_nll_kernel(B: int, C: int):
    num_subcores = 16
    b_per_w = B // num_subcores
    chunks = b_per_w // _LANES
    mesh = plsc.VectorSubcoreMesh(
        core_axis_name="c", subcore_axis_name="s", num_cores=1
    )

    @functools.partial(
        pl.kernel,
        out_type=jax.ShapeDtypeStruct((_LANES,), jnp.float32),
        mesh=mesh,
        compiler_params=pltpu.CompilerParams(needs_layout_passes=False),
        scratch_types=[
            pltpu.VMEM((b_per_w,), jnp.int32),
            pltpu.VMEM((b_per_w,), jnp.int32),
            pltpu.VMEM((b_per_w, 8, 128), jnp.float32),
            pltpu.VMEM((2 * _LANES,), jnp.float32),
            pltpu.VMEM_SHARED((num_subcores * 2 * _LANES,), jnp.float32),
            pltpu.VMEM((num_subcores * 2 * _LANES,), jnp.float32),
            pltpu.VMEM((_LANES,), jnp.float32),
            pltpu.VMEM((_LANES,), jnp.float32),
            pltpu.SemaphoreType.DMA,
        ],
    )
    def nll_kernel(predsT_hbm, tgt_hbm, out_hbm,
                   tgt_v, idx_v, slab_v, part_v, shared, all_v, res_v,
                   bfly_v, sem):
        sid = lax.axis_index("s")
        base = sid * b_per_w

        pltpu.sync_copy(tgt_hbm.at[pl.ds(base, b_per_w)], tgt_v)

        lane = lax.iota(jnp.int32, _LANES)
        for j in range(chunks):
            t = tgt_v[pl.ds(j * _LANES, _LANES)]
            safe = jnp.minimum(jnp.maximum(t, 0), C - 1)
            idx_v[pl.ds(j * _LANES, _LANES)] = safe >> 3

        c0 = pl.multiple_of((base >> 7) << 7, 128)
        view3 = predsT_hbm.reshape(C // 8, 8, B)
        pltpu.async_copy(
            view3.at[idx_v, :, pl.ds(c0, 128)], slab_v, sem
        ).wait()

        acc = jnp.zeros((_LANES,), jnp.float32)
        cnt = jnp.zeros((_LANES,), jnp.float32)
        for j in range(chunks):
            t = tgt_v[pl.ds(j * _LANES, _LANES)]
            valid = t != _IGNORE_INDEX
            safe = jnp.minimum(jnp.maximum(t, 0), C - 1)
            sample = j * _LANES + lane
            colw = (base + sample) & 127
            v = plsc.load_gather(slab_v, [sample, safe & 7, colw])
            acc = acc + jnp.where(valid, -v, 0.0)
            cnt = cnt + jnp.where(valid, 1.0, 0.0)

        part_v[pl.ds(0, _LANES)] = acc
        part_v[pl.ds(_LANES, _LANES)] = cnt
        pltpu.sync_copy(part_v, shared.at[pl.ds(sid * 2 * _LANES, 2 * _LANES)])
        plsc.subcore_barrier()

        pltpu.sync_copy(shared, all_v)
        tot = jnp.zeros((_LANES,), jnp.float32)
        num = jnp.zeros((_LANES,), jnp.float32)
        for w in range(num_subcores):
            tot = tot + all_v[pl.ds(w * 2 * _LANES, _LANES)]
            num = num + all_v[pl.ds(w * 2 * _LANES + _LANES, _LANES)]

        def lane_sum(vec):
            for shift in (8, 4, 2, 1):
                bfly_v[...] = vec
                vec = vec + plsc.load_gather(bfly_v, [lane ^ shift])
            return vec

        s = lane_sum(tot)
        n = lane_sum(num)
        res_v[...] = s / jnp.maximum(n, 1.0)

        @pl.when(sid == 0)
        def _():
            pltpu.sync_copy(res_v, out_hbm)

    return nll_kernel


def kernel(predictions, targets):
    B, C = predictions.shape
    tgt = targets.astype(jnp.int32)
    out = _make_nll_kernel(B, C)(predictions.T, tgt)
    return out[0]
